# Initial kernel scaffold; baseline (speedup 1.0000x reference)
import functools
import jax
import jax.numpy as jnp
import numpy as np
from jax.experimental import pallas as pl
from jax.experimental.pallas import tpu as pltpu

N = 10000
E = 320000
C = 32
K = 16
NBLOCKS = 4
LEVELS = 3
NPAD = 10240


_MLP_BLOCKS = 8
_RB = NPAD // _MLP_BLOCKS


def _mm_stats_kernel(x_ref, w_ref, o_ref, part_ref):
    h = jnp.dot(x_ref[...], w_ref[...], preferred_element_type=jnp.float32)
    o_ref[...] = h
    part_ref[0, 0, :] = jnp.sum(h, axis=0)
    part_ref[0, 1, :] = jnp.sum(h * h, axis=0)


def _bn_mm_stats_masked_kernel(x_ref, part_in_ref, w_ref, o_ref, part_ref, *, nrows):
    i = pl.program_id(0)
    s = jnp.sum(part_in_ref[...], axis=0)
    m = s[0] / nrows
    v = s[1] / nrows - m * m
    r = jax.lax.rsqrt(v + 1e-5)
    xn = jnp.maximum((x_ref[...] - m) * r, 0.0)
    row = jax.lax.broadcasted_iota(jnp.int32, xn.shape, 0) + i * _RB
    xn = jnp.where(row < nrows, xn, 0.0)
    h = jnp.dot(xn, w_ref[...], preferred_element_type=jnp.float32)
    o_ref[...] = h
    part_ref[0, 0, :] = jnp.sum(h, axis=0)
    part_ref[0, 1, :] = jnp.sum(h * h, axis=0)


def _bn_mm_masked_kernel(x_ref, part_in_ref, w_ref, o_ref, *, nrows):
    i = pl.program_id(0)
    s = jnp.sum(part_in_ref[...], axis=0)
    m = s[0] / nrows
    v = s[1] / nrows - m * m
    r = jax.lax.rsqrt(v + 1e-5)
    xn = jnp.maximum((x_ref[...] - m) * r, 0.0)
    row = jax.lax.broadcasted_iota(jnp.int32, xn.shape, 0) + i * _RB
    xn = jnp.where(row < nrows, xn, 0.0)
    o_ref[...] = jnp.dot(xn, w_ref[...], preferred_element_type=jnp.float32)


def _mm_stats(x, w):
    cin, cout = w.shape
    return pl.pallas_call(
        _mm_stats_kernel,
        grid=(_MLP_BLOCKS,),
        in_specs=[
            pl.BlockSpec((_RB, cin), lambda i: (i, 0)),
            pl.BlockSpec((cin, cout), lambda i: (0, 0)),
        ],
        out_specs=[
            pl.BlockSpec((_RB, cout), lambda i: (i, 0)),
            pl.BlockSpec((1, 2, cout), lambda i: (i, 0, 0)),
        ],
        out_shape=[
            jax.ShapeDtypeStruct((NPAD, cout), jnp.float32),
            jax.ShapeDtypeStruct((_MLP_BLOCKS, 2, cout), jnp.float32),
        ],
    )(x, w)


def _bn_mm_stats_masked(x, part_in, w, nrows):
    cin, cout = w.shape
    return pl.pallas_call(
        functools.partial(_bn_mm_stats_masked_kernel, nrows=nrows),
        grid=(_MLP_BLOCKS,),
        in_specs=[
            pl.BlockSpec((_RB, cin), lambda i: (i, 0)),
            pl.BlockSpec((_MLP_BLOCKS, 2, cin), lambda i: (0, 0, 0)),
            pl.BlockSpec((cin, cout), lambda i: (0, 0)),
        ],
        out_specs=[
            pl.BlockSpec((_RB, cout), lambda i: (i, 0)),
            pl.BlockSpec((1, 2, cout), lambda i: (i, 0, 0)),
        ],
        out_shape=[
            jax.ShapeDtypeStruct((NPAD, cout), jnp.float32),
            jax.ShapeDtypeStruct((_MLP_BLOCKS, 2, cout), jnp.float32),
        ],
    )(x, part_in, w)


def _bn_mm_masked(x, part_in, w, nrows):
    cin, cout = w.shape
    return pl.pallas_call(
        functools.partial(_bn_mm_masked_kernel, nrows=nrows),
        grid=(_MLP_BLOCKS,),
        in_specs=[
            pl.BlockSpec((_RB, cin), lambda i: (i, 0)),
            pl.BlockSpec((_MLP_BLOCKS, 2, cin), lambda i: (0, 0, 0)),
            pl.BlockSpec((cin, cout), lambda i: (0, 0)),
        ],
        out_specs=pl.BlockSpec((_RB, cout), lambda i: (i, 0)),
        out_shape=jax.ShapeDtypeStruct((NPAD, cout), jnp.float32),
    )(x, part_in, w)


def _mlp_head(fusion, p):
    h1, p1 = _mm_stats(fusion, p["fus_w"].T)
    h2, p2 = _bn_mm_stats_masked(h1, p1, p["p1_w"].T, N)
    h3, p3 = _bn_mm_stats_masked(h2, p2, p["p2_w"].T, N)
    out = _bn_mm_masked(h3, p3, jnp.pad(p["p3_w"], ((0, 115), (0, 0))).T, N)
    return out[:N, :13]


def _bn(h, g, b, eps=1e-5):
    m = jnp.mean(h, axis=0)
    v = jnp.var(h, axis=0)
    return (h - m) / jnp.sqrt(v + eps) * g + b


def _knn_idx(x, k):
    sq = jnp.sum(x * x, axis=1)
    cs = 2000
    outs = []
    for s in range(0, x.shape[0], cs):
        xq = x[s:s + cs]
        d = jnp.sum(xq * xq, axis=1)[:, None] - 2.0 * (xq @ x.T) + sq[None, :]
        d = d.at[jnp.arange(xq.shape[0]), jnp.arange(s, s + xq.shape[0])].set(jnp.inf)
        _, idx = jax.lax.top_k(-d, k)
        outs.append(idx)
    return jnp.concatenate(outs, axis=0)


def _edgeconv_knn(x, idx, w, b, g, be):
    xj = x[idx]
    xi = jnp.broadcast_to(x[:, None, :], xj.shape)
    msg = jnp.concatenate([xi, xj - xi], axis=-1).reshape(-1, 2 * x.shape[1])
    h = jax.nn.relu(_bn(msg @ w.T + b, g, be))
    return h.reshape(x.shape[0], idx.shape[1], -1).max(axis=1)


def _head_conv(x, edge_index, w, b, g, be):
    src = edge_index[0]
    dst = edge_index[1]
    xi = x[dst]
    xj = x[src]
    msg = jnp.concatenate([xi, xj - xi], axis=-1)
    h = jax.nn.relu(_bn(msg @ w.T + b, g, be))
    out = jax.ops.segment_max(h, dst, num_segments=x.shape[0])
    return jnp.where(jnp.isfinite(out), out, 0.0)


def _bifpn(feats, w1, w2, cw, cb, eps=1e-4):
    w1 = jax.nn.relu(w1); w1 = w1 / (jnp.sum(w1, axis=0) + eps)
    w2 = jax.nn.relu(w2); w2 = w2 / (jnp.sum(w2, axis=0) + eps)
    levels = len(feats)
    path = list(feats)
    orig = list(feats)
    idx = 0
    for i in range(levels - 1, 0, -1):
        fused = w1[0, i - 1] * path[i - 1] + w1[1, i - 1] * path[i]
        path[i - 1] = fused @ cw[idx].T + cb[idx]; idx += 1
    for i in range(0, levels - 2):
        fused = w2[0, i] * path[i + 1] + w2[1, i] * path[i] + w2[2, i] * orig[i + 1]
        path[i + 1] = fused @ cw[idx].T + cb[idx]; idx += 1
    fused = w1[0, levels - 1] * path[levels - 1] + w1[1, levels - 1] * path[levels - 2]
    path[levels - 1] = fused @ cw[idx].T + cb[idx]
    return path


def kernel(x, edge_index, edge_attr, params):
    p = params
    feats = [_head_conv(x, edge_index, p["head_w"], p["head_b"], p["head_g"], p["head_be"])]
    for i in range(NBLOCKS):
        h = feats[-1]
        idx = _knn_idx(h, K)
        feats.append(h + _edgeconv_knn(h, idx, p["blk_w"][i], p["blk_b"][i], p["blk_g"][i], p["blk_be"][i]))
    fusion = _bifpn(feats[len(feats) - LEVELS:], p["w1"], p["w2"], p["bifpn_w"], p["bifpn_b"])
    fusion = jnp.concatenate(fusion, axis=1)
    fusion = jnp.pad(fusion, ((0, NPAD - N), (0, 0)))
    return _mlp_head(fusion, p)


# speedup vs baseline: 1.1690x; 1.1690x over previous
import functools
import jax
import jax.numpy as jnp
from jax.experimental import pallas as pl

N = 10000
E = 320000
C = 32
K = 16
NBLOCKS = 4
LEVELS = 3
NPAD = 10240

_DEF = jax.lax.Precision.DEFAULT


def _pack_stats(h):
    s0 = jnp.sum(h, axis=0, keepdims=True)
    s1 = jnp.sum(h * h, axis=0, keepdims=True)
    r = jax.lax.broadcasted_iota(jnp.int32, (8, h.shape[1]), 0)
    return jnp.where(r == 0, s0, jnp.where(r == 1, s1, 0.0))


_QB = 400


def _dist_kernel(x_ref, sqr_ref, sqc_ref, q_ref, d_ref):
    i = pl.program_id(0)
    xq = q_ref[...]
    mm = jax.lax.dot_general(xq, x_ref[...], (((1,), (1,)), ((), ())),
                             precision=_DEF, preferred_element_type=jnp.float32)
    sqq = sqr_ref[0, 0, :]
    sq = sqc_ref[0, :]
    d = sqq[:, None] - 2.0 * mm + sq[None, :]
    row = jax.lax.broadcasted_iota(jnp.int32, d.shape, 0) + i * _QB
    col = jax.lax.broadcasted_iota(jnp.int32, d.shape, 1)
    d_ref[...] = jnp.where(row == col, jnp.inf, d)


def _pairdist(x, sq):
    return pl.pallas_call(
        _dist_kernel,
        grid=(N // _QB,),
        in_specs=[
            pl.BlockSpec((N, C), lambda i: (0, 0)),
            pl.BlockSpec((1, 1, _QB), lambda i: (i, 0, 0)),
            pl.BlockSpec((1, N), lambda i: (0, 0)),
            pl.BlockSpec((_QB, C), lambda i: (i, 0)),
        ],
        out_specs=pl.BlockSpec((_QB, N), lambda i: (i, 0)),
        out_shape=jax.ShapeDtypeStruct((N, N), jnp.float32),
    )(x, sq.reshape(N // _QB, 1, _QB), sq.reshape(1, N), x)


def _knn_idx_pallas(x):
    sq = jnp.sum(x * x, axis=1)
    d = _pairdist(x, sq)
    _, idx = jax.lax.top_k(-d, K)
    return idx


_EB = 3200


def _ecmsg_kernel(msg_ref, w_ref, h_ref, mx_ref):
    h = jax.lax.dot_general(msg_ref[...], w_ref[...], (((1,), (1,)), ((), ())),
                            precision=_DEF, preferred_element_type=jnp.float32)
    h_ref[...] = h
    mx_ref[...] = jnp.max(h.reshape(_EB // K, K, C), axis=1)


def _ecmsg(msg, w):
    nb = (N * K) // _EB
    return pl.pallas_call(
        _ecmsg_kernel,
        grid=(nb,),
        in_specs=[
            pl.BlockSpec((_EB, 2 * C), lambda i: (i, 0)),
            pl.BlockSpec((C, 2 * C), lambda i: (0, 0)),
        ],
        out_specs=[
            pl.BlockSpec((_EB, C), lambda i: (i, 0)),
            pl.BlockSpec((_EB // K, C), lambda i: (i, 0)),
        ],
        out_shape=[
            jax.ShapeDtypeStruct((N * K, C), jnp.float32),
            jax.ShapeDtypeStruct((N, C), jnp.float32),
        ],
    )(msg, w)


def _edgeconv_block(x, idx, w, g, b):
    xj = x[idx]
    xi = jnp.broadcast_to(x[:, None, :], xj.shape)
    msg = jnp.concatenate([xi, xj - xi], axis=-1).reshape(-1, 2 * C)
    h, mx = _ecmsg(msg, w)
    m = jnp.mean(h, axis=0)
    v = jnp.var(h, axis=0)
    out = jax.nn.relu((mx - m) / jnp.sqrt(v + 1e-5) * g + b)
    return x + out


_HB = 3200


def _headmsg_kernel(msg_ref, w_ref, h_ref):
    h_ref[...] = jax.lax.dot_general(msg_ref[...], w_ref[...], (((1,), (1,)), ((), ())),
                                     precision=_DEF, preferred_element_type=jnp.float32)


def _headmsg(msg, w):
    nb = E // _HB
    return pl.pallas_call(
        _headmsg_kernel,
        grid=(nb,),
        in_specs=[
            pl.BlockSpec((_HB, 2 * C), lambda i: (i, 0)),
            pl.BlockSpec((C, 2 * C), lambda i: (0, 0)),
        ],
        out_specs=pl.BlockSpec((_HB, C), lambda i: (i, 0)),
        out_shape=jax.ShapeDtypeStruct((E, C), jnp.float32),
    )(msg, w)


def _head_conv_p(x, edge_index, w, g, b):
    src = edge_index[0]
    dst = edge_index[1]
    xi = x[dst]
    xj = x[src]
    msg = jnp.concatenate([xi, xj - xi], axis=-1)
    h = _headmsg(msg, w)
    m = jnp.mean(h, axis=0)
    v = jnp.var(h, axis=0)
    mx = jax.ops.segment_max(h, dst, num_segments=N)
    out = jax.nn.relu((mx - m) / jnp.sqrt(v + 1e-5) * g + b)
    return jnp.where(jnp.isfinite(out), out, 0.0)


_MLP_BLOCKS = 8
_RB = NPAD // _MLP_BLOCKS


def _unpack_stats(part, nrows):
    s = jnp.sum(part, axis=0)
    m = s[0] / nrows
    v = s[1] / nrows - m * m
    return m, jax.lax.rsqrt(v + 1e-5)


def _mm_stats_kernel(x_ref, w_ref, o_ref, part_ref):
    h = jnp.dot(x_ref[...], w_ref[...], preferred_element_type=jnp.float32,
                precision=_DEF)
    o_ref[...] = h
    part_ref[0, :, :] = _pack_stats(h)


def _bn_mm_stats_masked_kernel(x_ref, part_in_ref, w_ref, o_ref, part_ref, *, nrows):
    i = pl.program_id(0)
    m, r = _unpack_stats(part_in_ref[...], nrows)
    xn = jnp.maximum((x_ref[...] - m) * r, 0.0)
    row = jax.lax.broadcasted_iota(jnp.int32, xn.shape, 0) + i * _RB
    xn = jnp.where(row < nrows, xn, 0.0)
    h = jnp.dot(xn, w_ref[...], preferred_element_type=jnp.float32,
                precision=_DEF)
    o_ref[...] = h
    part_ref[0, :, :] = _pack_stats(h)


def _bn_mm_masked_kernel(x_ref, part_in_ref, w_ref, o_ref, *, nrows):
    i = pl.program_id(0)
    m, r = _unpack_stats(part_in_ref[...], nrows)
    xn = jnp.maximum((x_ref[...] - m) * r, 0.0)
    row = jax.lax.broadcasted_iota(jnp.int32, xn.shape, 0) + i * _RB
    xn = jnp.where(row < nrows, xn, 0.0)
    o_ref[...] = jnp.dot(xn, w_ref[...], preferred_element_type=jnp.float32,
                         precision=_DEF)


def _mm_stats(x, w):
    cin, cout = w.shape
    return pl.pallas_call(
        _mm_stats_kernel,
        grid=(_MLP_BLOCKS,),
        in_specs=[
            pl.BlockSpec((_RB, cin), lambda i: (i, 0)),
            pl.BlockSpec((cin, cout), lambda i: (0, 0)),
        ],
        out_specs=[
            pl.BlockSpec((_RB, cout), lambda i: (i, 0)),
            pl.BlockSpec((1, 8, cout), lambda i: (i, 0, 0)),
        ],
        out_shape=[
            jax.ShapeDtypeStruct((NPAD, cout), jnp.float32),
            jax.ShapeDtypeStruct((_MLP_BLOCKS, 8, cout), jnp.float32),
        ],
    )(x, w)


def _bn_mm_stats_masked(x, part_in, w, nrows):
    cin, cout = w.shape
    return pl.pallas_call(
        functools.partial(_bn_mm_stats_masked_kernel, nrows=nrows),
        grid=(_MLP_BLOCKS,),
        in_specs=[
            pl.BlockSpec((_RB, cin), lambda i: (i, 0)),
            pl.BlockSpec((_MLP_BLOCKS, 8, cin), lambda i: (0, 0, 0)),
            pl.BlockSpec((cin, cout), lambda i: (0, 0)),
        ],
        out_specs=[
            pl.BlockSpec((_RB, cout), lambda i: (i, 0)),
            pl.BlockSpec((1, 8, cout), lambda i: (i, 0, 0)),
        ],
        out_shape=[
            jax.ShapeDtypeStruct((NPAD, cout), jnp.float32),
            jax.ShapeDtypeStruct((_MLP_BLOCKS, 8, cout), jnp.float32),
        ],
    )(x, part_in, w)


def _bn_mm_masked(x, part_in, w, nrows):
    cin, cout = w.shape
    return pl.pallas_call(
        functools.partial(_bn_mm_masked_kernel, nrows=nrows),
        grid=(_MLP_BLOCKS,),
        in_specs=[
            pl.BlockSpec((_RB, cin), lambda i: (i, 0)),
            pl.BlockSpec((_MLP_BLOCKS, 8, cin), lambda i: (0, 0, 0)),
            pl.BlockSpec((cin, cout), lambda i: (0, 0)),
        ],
        out_specs=pl.BlockSpec((_RB, cout), lambda i: (i, 0)),
        out_shape=jax.ShapeDtypeStruct((NPAD, cout), jnp.float32),
    )(x, part_in, w)


def _mlp_head(fusion, p):
    h1, p1 = _mm_stats(fusion, p["fus_w"].T)
    h2, p2 = _bn_mm_stats_masked(h1, p1, p["p1_w"].T, N)
    h3, p3 = _bn_mm_stats_masked(h2, p2, p["p2_w"].T, N)
    out = _bn_mm_masked(h3, p3, jnp.pad(p["p3_w"], ((0, 115), (0, 0))).T, N)
    return out[:N, :13]


def _bifpn(feats, w1, w2, cw, cb, eps=1e-4):
    w1 = jax.nn.relu(w1); w1 = w1 / (jnp.sum(w1, axis=0) + eps)
    w2 = jax.nn.relu(w2); w2 = w2 / (jnp.sum(w2, axis=0) + eps)
    levels = len(feats)
    path = list(feats)
    orig = list(feats)
    idx = 0
    for i in range(levels - 1, 0, -1):
        fused = w1[0, i - 1] * path[i - 1] + w1[1, i - 1] * path[i]
        path[i - 1] = fused @ cw[idx].T + cb[idx]; idx += 1
    for i in range(0, levels - 2):
        fused = w2[0, i] * path[i + 1] + w2[1, i] * path[i] + w2[2, i] * orig[i + 1]
        path[i + 1] = fused @ cw[idx].T + cb[idx]; idx += 1
    fused = w1[0, levels - 1] * path[levels - 1] + w1[1, levels - 1] * path[levels - 2]
    path[levels - 1] = fused @ cw[idx].T + cb[idx]
    return path


def kernel(x, edge_index, edge_attr, params):
    p = params
    feats = [_head_conv_p(x, edge_index, p["head_w"], p["head_g"], p["head_be"])]
    for i in range(NBLOCKS):
        h = feats[-1]
        idx = _knn_idx_pallas(h)
        feats.append(_edgeconv_block(h, idx, p["blk_w"][i], p["blk_g"][i], p["blk_be"][i]))
    fusion = _bifpn(feats[len(feats) - LEVELS:], p["w1"], p["w2"], p["bifpn_w"], p["bifpn_b"])
    fusion = jnp.concatenate(fusion, axis=1)
    fusion = jnp.pad(fusion, ((0, NPAD - N), (0, 0)))
    return _mlp_head(fusion, p)
